# baseline (speedup 1.0000x reference)
import functools
import math

import jax
import jax.numpy as jnp
from jax import lax
from jax.experimental import pallas as pl
from jax.experimental.pallas import tpu as pltpu
from jax.experimental.pallas import tpu_sc as plsc

VOCAB = 1000000
EMB = 64
SCALE = math.sqrt(EMB)

_NUM_CORES = 2
_NUM_SUBCORES = 16
_NW = _NUM_CORES * _NUM_SUBCORES

_B = 4096 * 200
_B_PER_W = _B // _NW
_CHUNK = 512
_NCHUNKS = _B_PER_W // _CHUNK


def _sc_embed(tokens_flat, table):
    mesh = plsc.VectorSubcoreMesh(
        core_axis_name="c", subcore_axis_name="s")

    @functools.partial(
        pl.kernel,
        out_type=jax.ShapeDtypeStruct((_B, EMB), jnp.float32),
        mesh=mesh,
        scratch_types=[
            pltpu.VMEM((_CHUNK,), jnp.int32),
            pltpu.VMEM((_CHUNK, EMB), jnp.float32),
            pltpu.SemaphoreType.DMA,
        ],
        compiler_params=pltpu.CompilerParams(use_tc_tiling_on_sc=False),
    )
    def body(tok_hbm, table_hbm, out_hbm, idx_v, rows_v, sem):
        wid = lax.axis_index("s") * _NUM_CORES + lax.axis_index("c")
        base = wid * _B_PER_W

        def chunk(g, carry):
            off = base + g * _CHUNK
            pltpu.sync_copy(tok_hbm.at[pl.ds(off, _CHUNK)], idx_v)
            pltpu.async_copy(table_hbm.at[idx_v], rows_v, sem).wait()

            def scale_row(i, c2):
                for j in range(EMB // 16):
                    sl = pl.ds(j * 16, 16)
                    rows_v[i, sl] = rows_v[i, sl] * SCALE
                return c2

            lax.fori_loop(0, _CHUNK, scale_row, 0)
            pltpu.sync_copy(rows_v, out_hbm.at[pl.ds(off, _CHUNK)])
            return carry

        lax.fori_loop(0, _NCHUNKS, chunk, 0)

    return body(tokens_flat, table)


def kernel(tokens, table):
    tokens_flat = tokens.reshape(-1).astype(jnp.int32)
    out = _sc_embed(tokens_flat, table)
    return out.reshape(tokens.shape + (EMB,))


# speedup vs baseline: 1.1357x; 1.1357x over previous
import functools
import math

import jax
import jax.numpy as jnp
from jax import lax
from jax.experimental import pallas as pl
from jax.experimental.pallas import tpu as pltpu
from jax.experimental.pallas import tpu_sc as plsc

VOCAB = 1000000
EMB = 64
SCALE = math.sqrt(EMB)

_NUM_CORES = 2
_NUM_SUBCORES = 16
_NW = _NUM_CORES * _NUM_SUBCORES

_B = 4096 * 200
_B_PER_W = _B // _NW
_CHUNK = 256
_NCHUNKS = _B_PER_W // _CHUNK
_NBUF = 4
_LOOKAHEAD = 2


def _sc_embed(tokens_flat, table):
    mesh = plsc.VectorSubcoreMesh(
        core_axis_name="c", subcore_axis_name="s")

    @functools.partial(
        pl.kernel,
        out_type=jax.ShapeDtypeStruct((_B, EMB), jnp.float32),
        mesh=mesh,
        scratch_types=[
            pltpu.VMEM((_NCHUNKS, _CHUNK), jnp.int32),
            [pltpu.VMEM((_CHUNK, EMB), jnp.float32)] * _NBUF,
            [pltpu.SemaphoreType.DMA] * _NBUF,
            [pltpu.SemaphoreType.DMA] * _NBUF,
        ],
        compiler_params=pltpu.CompilerParams(use_tc_tiling_on_sc=False),
    )
    def body(tok_hbm, table_hbm, out_hbm, idx_all, rows, gsem, wsem):
        wid = lax.axis_index("s") * _NUM_CORES + lax.axis_index("c")
        base = wid * _B_PER_W

        pltpu.sync_copy(tok_hbm.at[wid], idx_all)

        def gather(g, b):
            return pltpu.make_async_copy(
                table_hbm.at[idx_all.at[g]], rows[b], gsem[b])

        def write(g, b):
            return pltpu.make_async_copy(
                rows[b], out_hbm.at[pl.ds(base + g * _CHUNK, _CHUNK)],
                wsem[b])

        def scale(b):
            r = rows[b]

            @plsc.parallel_loop(0, _CHUNK, unroll=4)
            def _(i):
                for j in range(EMB // 16):
                    sl = pl.ds(j * 16, 16)
                    r[i, sl] = r[i, sl] * SCALE

        def step(g, p, wait_write, prefetch):
            gather(g, p).wait()
            scale(p)
            write(g, p).start()
            if prefetch:
                f = g + _LOOKAHEAD
                q = (p + _LOOKAHEAD) % _NBUF
                if wait_write:
                    write(f - _NBUF, q).wait()
                gather(f, q).start()

        for j in range(_LOOKAHEAD):
            gather(j, j).start()
        for g in range(_NBUF - _LOOKAHEAD):
            step(g, g % _NBUF, wait_write=False, prefetch=True)
        head = _NBUF - _LOOKAHEAD
        nblocks = (_NCHUNKS - _NBUF) // _NBUF

        def block(G, carry):
            for b in range(_NBUF):
                g = head + G * _NBUF + b
                step(g, (head + b) % _NBUF, wait_write=True, prefetch=True)
            return carry

        lax.fori_loop(0, nblocks, block, 0)
        for g in range(_NCHUNKS - _LOOKAHEAD, _NCHUNKS):
            step(g, g % _NBUF, wait_write=False, prefetch=False)
        for b in range(_NBUF):
            g = _NCHUNKS - _NBUF + b
            write(g, g % _NBUF).wait()

    return body(tokens_flat, table)


def kernel(tokens, table):
    tokens_w = tokens.reshape(_NW, _NCHUNKS, _CHUNK).astype(jnp.int32)
    out = _sc_embed(tokens_w, table)
    return out.reshape(tokens.shape + (EMB,))
